# Initial kernel scaffold; baseline (speedup 1.0000x reference)
import jax
import jax.numpy as jnp
from jax.experimental import pallas as pl


def kernel(x, edge_index, W1, b1, W2, b2):
    raise NotImplementedError("write your pallas kernel here")


# speedup vs baseline: 8.6498x; 8.6498x over previous
import functools

import jax
import jax.numpy as jnp
from jax import lax
from jax.experimental import pallas as pl
from jax.experimental.pallas import tpu as pltpu
from jax.experimental.pallas import tpu_sc as plsc

N = 10000
E = 160000
D = 256
HALF = D // 2
ALPHA = 0.5

NC = 2
NS = 16
NW = NC * NS

EDGES_PER_TILE_DEG = E // NW
EDGES_PER_TILE = E // NS
CHUNK = 80
NCHUNKS = EDGES_PER_TILE // CHUNK
ROWS_PER_TILE = N // NS

_mesh = plsc.VectorSubcoreMesh(core_axis_name="c", subcore_axis_name="s")


@functools.partial(
    pl.kernel,
    out_type=jax.ShapeDtypeStruct((NW, N), jnp.float32),
    mesh=_mesh,
    scratch_types=[
        pltpu.VMEM((EDGES_PER_TILE_DEG,), jnp.int32),
        pltpu.VMEM((N,), jnp.float32),
    ],
    compiler_params=pltpu.CompilerParams(needs_layout_passes=False),
)
def _sc_degree(col_hbm, out_hbm, cidx, hist):
    c = lax.axis_index("c")
    s = lax.axis_index("s")
    wid = s * NC + c

    def zero(i, _):
        hist[pl.ds(i * 16, 16)] = jnp.zeros((16,), jnp.float32)
        return 0

    lax.fori_loop(0, N // 16, zero, 0)

    pltpu.sync_copy(col_hbm.at[pl.ds(wid * EDGES_PER_TILE_DEG, EDGES_PER_TILE_DEG)], cidx)

    ones = jnp.ones((16,), jnp.float32)
    nfull = EDGES_PER_TILE_DEG // 16
    rem = EDGES_PER_TILE_DEG - nfull * 16

    def upd(k, _):
        cv = cidx[pl.ds(k * 16, 16)]
        plsc.addupdate_scatter(hist, [cv], ones)
        return 0

    lax.fori_loop(0, nfull, upd, 0)

    lane = lax.iota(jnp.int32, 16)
    cv = cidx[pl.ds(EDGES_PER_TILE_DEG - 16, 16)]
    keep = lane >= (16 - rem)
    plsc.addupdate_scatter(
        hist,
        [jnp.where(keep, cv, 0)],
        jnp.where(keep, 1.0, 0.0).astype(jnp.float32),
    )

    pltpu.sync_copy(hist, out_hbm.at[wid])


@functools.partial(
    pl.kernel,
    out_type=jax.ShapeDtypeStruct((NC * N, HALF), jnp.float32),
    mesh=_mesh,
    scratch_types=[
        pltpu.VMEM_SHARED((N, HALF), jnp.float32),
        pltpu.VMEM((CHUNK, HALF), jnp.float32),
        pltpu.VMEM((CHUNK,), jnp.int32),
        pltpu.VMEM((CHUNK,), jnp.int32),
        pltpu.SemaphoreType.DMA,
    ],
    compiler_params=pltpu.CompilerParams(use_tc_tiling_on_sc=False),
)
def _sc_propagate(u_hbm, row_hbm, col_hbm, out_hbm, slab, gbuf, ridx, cidx, sem):
    c = lax.axis_index("c")
    s = lax.axis_index("s")

    pltpu.sync_copy(
        u_hbm.at[pl.ds(c * N + s * ROWS_PER_TILE, ROWS_PER_TILE)],
        slab.at[pl.ds(s * ROWS_PER_TILE, ROWS_PER_TILE)],
    )
    plsc.subcore_barrier()

    base0 = s * EDGES_PER_TILE
    roff = c * N

    def chunk(j, _):
        base = base0 + j * CHUNK
        pltpu.sync_copy(row_hbm.at[pl.ds(base, CHUNK)], ridx)
        pltpu.sync_copy(col_hbm.at[pl.ds(base, CHUNK)], cidx)
        for k in range(CHUNK // 16):
            ridx[pl.ds(k * 16, 16)] = ridx[pl.ds(k * 16, 16)] + roff
        pltpu.async_copy(u_hbm.at[ridx], gbuf, sem).wait()
        pltpu.sync_copy(gbuf, slab.at[cidx], add=True)
        return 0

    lax.fori_loop(0, NCHUNKS, chunk, 0)
    plsc.subcore_barrier()

    pltpu.sync_copy(
        slab.at[pl.ds(s * ROWS_PER_TILE, ROWS_PER_TILE)],
        out_hbm.at[pl.ds(c * N + s * ROWS_PER_TILE, ROWS_PER_TILE)],
    )


_BLK = 2000
_GRID = N // _BLK


def _tc_dis_body(degp_ref, dis_ref):
    deg = jnp.sum(degp_ref[...], axis=0) + 1.0
    dis_ref[...] = lax.rsqrt(deg).reshape(N, 1)


def _tc_scale0_body(dis_ref, x_ref, out_ref):
    dis = dis_ref[...]
    xb = x_ref[...]
    out_ref[...] = jnp.stack(
        [dis * xb[:, :HALF], dis * xb[:, HALF:]], axis=0
    )


def _tc_scale1_body(dis_ref, s_ref, x_ref, out_ref):
    dis = dis_ref[...]
    sb = s_ref[...]
    xb = x_ref[...]
    a = (1.0 - ALPHA) * dis * dis
    b = ALPHA * dis
    out_ref[...] = jnp.stack(
        [a * sb[0] + b * xb[:, :HALF], a * sb[1] + b * xb[:, HALF:]], axis=0
    )


def _tc_final_body(dis_ref, s_ref, x_ref, wc_ref, bc_ref, out_ref):
    dis = dis_ref[...]
    sb = s_ref[...]
    xb = x_ref[...]
    agg = jnp.concatenate([sb[0], sb[1]], axis=-1)
    y = jax.nn.relu((1.0 - ALPHA) * dis * agg + ALPHA * xb)
    out_ref[...] = (
        jnp.dot(y, wc_ref[...], preferred_element_type=jnp.float32) + bc_ref[...]
    )


def _tc_weights_body(w1_ref, b1_ref, w2_ref, b2_ref, wc_ref, bc_ref):
    w2 = w2_ref[...]
    wc_ref[...] = jnp.dot(w1_ref[...], w2, preferred_element_type=jnp.float32)
    bc_ref[...] = jnp.dot(b1_ref[...], w2, preferred_element_type=jnp.float32) + b2_ref[...]


_dis_spec = pl.BlockSpec((_BLK, 1), lambda i: (i, 0))
_x_spec = pl.BlockSpec((_BLK, D), lambda i: (i, 0))
_s_spec = pl.BlockSpec((NC, _BLK, HALF), lambda i: (0, i, 0))


def _tc_dis(degp):
    return pl.pallas_call(
        _tc_dis_body,
        out_shape=jax.ShapeDtypeStruct((N, 1), jnp.float32),
    )(degp)


def _tc_scale0(dis, x):
    return pl.pallas_call(
        _tc_scale0_body,
        grid=(_GRID,),
        in_specs=[_dis_spec, _x_spec],
        out_specs=_s_spec,
        out_shape=jax.ShapeDtypeStruct((NC, N, HALF), jnp.float32),
    )(dis, x)


def _tc_scale1(dis, s, x):
    return pl.pallas_call(
        _tc_scale1_body,
        grid=(_GRID,),
        in_specs=[_dis_spec, _s_spec, _x_spec],
        out_specs=_s_spec,
        out_shape=jax.ShapeDtypeStruct((NC, N, HALF), jnp.float32),
    )(dis, s, x)


def _tc_final(dis, s, x, wc, bc):
    return pl.pallas_call(
        _tc_final_body,
        grid=(_GRID,),
        in_specs=[
            _dis_spec,
            _s_spec,
            _x_spec,
            pl.BlockSpec((D, D), lambda i: (0, 0)),
            pl.BlockSpec((1, D), lambda i: (0, 0)),
        ],
        out_specs=_x_spec,
        out_shape=jax.ShapeDtypeStruct((N, D), jnp.float32),
    )(dis, s, x, wc, bc)


def _tc_weights(w1, b1, w2, b2):
    return pl.pallas_call(
        _tc_weights_body,
        out_shape=[
            jax.ShapeDtypeStruct((D, D), jnp.float32),
            jax.ShapeDtypeStruct((1, D), jnp.float32),
        ],
    )(w1, b1, w2, b2)


def kernel(x, edge_index, W1, b1, W2, b2):
    row = edge_index[0]
    col = edge_index[1]

    degp = _sc_degree(col)
    dis = _tc_dis(degp)
    u0 = _tc_scale0(dis, x).reshape(NC * N, HALF)
    s0 = _sc_propagate(u0, row, col).reshape(NC, N, HALF)
    u1 = _tc_scale1(dis, s0, x).reshape(NC * N, HALF)
    s1 = _sc_propagate(u1, row, col).reshape(NC, N, HALF)
    wc, bc = _tc_weights(W1, b1.reshape(1, D), W2, b2.reshape(1, D))
    return _tc_final(dis, s1, x, wc, bc)


# speedup vs baseline: 19.4019x; 2.2431x over previous
import functools

import jax
import jax.numpy as jnp
from jax import lax
from jax.experimental import pallas as pl
from jax.experimental.pallas import tpu as pltpu
from jax.experimental.pallas import tpu_sc as plsc

N = 10000
E = 160000
D = 256
HALF = D // 2
ALPHA = 0.5

NC = 2
NS = 16
NW = NC * NS

EDGES_PER_TILE_DEG = E // NW
EDGES_PER_TILE = E // NS
CHUNK = 80
NCHUNKS = EDGES_PER_TILE // CHUNK
ROWS_PER_TILE = N // NS

_mesh = plsc.VectorSubcoreMesh(core_axis_name="c", subcore_axis_name="s")


@functools.partial(
    pl.kernel,
    out_type=jax.ShapeDtypeStruct((NW, N), jnp.float32),
    mesh=_mesh,
    scratch_types=[
        pltpu.VMEM((EDGES_PER_TILE_DEG,), jnp.int32),
        pltpu.VMEM((N,), jnp.float32),
    ],
    compiler_params=pltpu.CompilerParams(needs_layout_passes=False),
)
def _sc_degree(col_hbm, out_hbm, cidx, hist):
    c = lax.axis_index("c")
    s = lax.axis_index("s")
    wid = s * NC + c

    def zero(i, _):
        hist[pl.ds(i * 16, 16)] = jnp.zeros((16,), jnp.float32)
        return 0

    lax.fori_loop(0, N // 16, zero, 0)

    pltpu.sync_copy(col_hbm.at[pl.ds(wid * EDGES_PER_TILE_DEG, EDGES_PER_TILE_DEG)], cidx)

    ones = jnp.ones((16,), jnp.float32)
    nfull = EDGES_PER_TILE_DEG // 16
    rem = EDGES_PER_TILE_DEG - nfull * 16

    def upd(k, _):
        cv = cidx[pl.ds(k * 16, 16)]
        plsc.addupdate_scatter(hist, [cv], ones)
        return 0

    lax.fori_loop(0, nfull, upd, 0)

    lane = lax.iota(jnp.int32, 16)
    cv = cidx[pl.ds(EDGES_PER_TILE_DEG - 16, 16)]
    keep = lane >= (16 - rem)
    plsc.addupdate_scatter(
        hist,
        [jnp.where(keep, cv, 0)],
        jnp.where(keep, 1.0, 0.0).astype(jnp.float32),
    )

    pltpu.sync_copy(hist, out_hbm.at[wid])


@functools.partial(
    pl.kernel,
    out_type=jax.ShapeDtypeStruct((NC * N, HALF), jnp.float32),
    mesh=_mesh,
    scratch_types=[
        pltpu.VMEM_SHARED((N, HALF), jnp.float32),
        pltpu.VMEM((CHUNK, HALF), jnp.float32),
        pltpu.VMEM((CHUNK, HALF), jnp.float32),
        pltpu.VMEM((NCHUNKS, CHUNK), jnp.int32),
        pltpu.VMEM((NCHUNKS, CHUNK), jnp.int32),
        pltpu.SemaphoreType.DMA,
        pltpu.SemaphoreType.DMA,
    ],
    compiler_params=pltpu.CompilerParams(use_tc_tiling_on_sc=False),
)
def _sc_propagate(u_hbm, row2d_hbm, col2d_hbm, out_hbm,
                  slab, gbufa, gbufb, ridx, cidx, sema, semb):
    c = lax.axis_index("c")
    s = lax.axis_index("s")

    pltpu.sync_copy(
        row2d_hbm.at[pl.ds((c * NS + s) * NCHUNKS, NCHUNKS)], ridx)
    pltpu.sync_copy(col2d_hbm.at[pl.ds(s * NCHUNKS, NCHUNKS)], cidx)
    pltpu.sync_copy(
        u_hbm.at[pl.ds(c * N + s * ROWS_PER_TILE, ROWS_PER_TILE)],
        slab.at[pl.ds(s * ROWS_PER_TILE, ROWS_PER_TILE)],
    )
    plsc.subcore_barrier()

    def gather(j, buf, sem):
        return pltpu.make_async_copy(u_hbm.at[ridx.at[j]], buf, sem)

    def scatter(j, buf):
        pltpu.sync_copy(buf, slab.at[cidx.at[j]], add=True)

    gather(0, gbufa, sema).start()

    def body(jj, _):
        c0 = 2 * jj
        gather(c0 + 1, gbufb, semb).start()
        gather(c0, gbufa, sema).wait()
        scatter(c0, gbufa)
        gather(c0 + 2, gbufa, sema).start()
        gather(c0 + 1, gbufb, semb).wait()
        scatter(c0 + 1, gbufb)
        return 0

    lax.fori_loop(0, (NCHUNKS - 1) // 2, body, 0)
    gather(NCHUNKS - 1, gbufa, sema).wait()
    scatter(NCHUNKS - 1, gbufa)
    plsc.subcore_barrier()

    pltpu.sync_copy(
        slab.at[pl.ds(s * ROWS_PER_TILE, ROWS_PER_TILE)],
        out_hbm.at[pl.ds(c * N + s * ROWS_PER_TILE, ROWS_PER_TILE)],
    )


_BLK = 2000
_GRID = N // _BLK


def _tc_dis_body(degp_ref, dis_ref):
    deg = jnp.sum(degp_ref[...], axis=0) + 1.0
    dis_ref[...] = lax.rsqrt(deg).reshape(N, 1)


def _tc_scale0_body(dis_ref, x_ref, out_ref):
    dis = dis_ref[...]
    xb = x_ref[...]
    out_ref[...] = jnp.stack(
        [dis * xb[:, :HALF], dis * xb[:, HALF:]], axis=0
    )


def _tc_scale1_body(dis_ref, s_ref, x_ref, out_ref):
    dis = dis_ref[...]
    sb = s_ref[...]
    xb = x_ref[...]
    a = (1.0 - ALPHA) * dis * dis
    b = ALPHA * dis
    out_ref[...] = jnp.stack(
        [a * sb[0] + b * xb[:, :HALF], a * sb[1] + b * xb[:, HALF:]], axis=0
    )


def _tc_final_body(dis_ref, s_ref, x_ref, wc_ref, bc_ref, out_ref):
    dis = dis_ref[...]
    sb = s_ref[...]
    xb = x_ref[...]
    agg = jnp.concatenate([sb[0], sb[1]], axis=-1)
    y = jax.nn.relu((1.0 - ALPHA) * dis * agg + ALPHA * xb)
    out_ref[...] = (
        jnp.dot(y, wc_ref[...], preferred_element_type=jnp.float32) + bc_ref[...]
    )


def _tc_weights_body(w1_ref, b1_ref, w2_ref, b2_ref, wc_ref, bc_ref):
    w2 = w2_ref[...]
    wc_ref[...] = jnp.dot(w1_ref[...], w2, preferred_element_type=jnp.float32)
    bc_ref[...] = jnp.dot(b1_ref[...], w2, preferred_element_type=jnp.float32) + b2_ref[...]


_dis_spec = pl.BlockSpec((_BLK, 1), lambda i: (i, 0))
_x_spec = pl.BlockSpec((_BLK, D), lambda i: (i, 0))
_s_spec = pl.BlockSpec((NC, _BLK, HALF), lambda i: (0, i, 0))


def _tc_dis(degp):
    return pl.pallas_call(
        _tc_dis_body,
        out_shape=jax.ShapeDtypeStruct((N, 1), jnp.float32),
    )(degp)


def _tc_scale0(dis, x):
    return pl.pallas_call(
        _tc_scale0_body,
        grid=(_GRID,),
        in_specs=[_dis_spec, _x_spec],
        out_specs=_s_spec,
        out_shape=jax.ShapeDtypeStruct((NC, N, HALF), jnp.float32),
    )(dis, x)


def _tc_scale1(dis, s, x):
    return pl.pallas_call(
        _tc_scale1_body,
        grid=(_GRID,),
        in_specs=[_dis_spec, _s_spec, _x_spec],
        out_specs=_s_spec,
        out_shape=jax.ShapeDtypeStruct((NC, N, HALF), jnp.float32),
    )(dis, s, x)


def _tc_final(dis, s, x, wc, bc):
    return pl.pallas_call(
        _tc_final_body,
        grid=(_GRID,),
        in_specs=[
            _dis_spec,
            _s_spec,
            _x_spec,
            pl.BlockSpec((D, D), lambda i: (0, 0)),
            pl.BlockSpec((1, D), lambda i: (0, 0)),
        ],
        out_specs=_x_spec,
        out_shape=jax.ShapeDtypeStruct((N, D), jnp.float32),
    )(dis, s, x, wc, bc)


def _tc_weights(w1, b1, w2, b2):
    return pl.pallas_call(
        _tc_weights_body,
        out_shape=[
            jax.ShapeDtypeStruct((D, D), jnp.float32),
            jax.ShapeDtypeStruct((1, D), jnp.float32),
        ],
    )(w1, b1, w2, b2)


def kernel(x, edge_index, W1, b1, W2, b2):
    row = edge_index[0]
    col = edge_index[1]

    row2d = jnp.concatenate([row, row + N]).reshape(NC * E // CHUNK, CHUNK)
    col2d = col.reshape(E // CHUNK, CHUNK)

    degp = _sc_degree(col)
    dis = _tc_dis(degp)
    u0 = _tc_scale0(dis, x).reshape(NC * N, HALF)
    s0 = _sc_propagate(u0, row2d, col2d).reshape(NC, N, HALF)
    u1 = _tc_scale1(dis, s0, x).reshape(NC * N, HALF)
    s1 = _sc_propagate(u1, row2d, col2d).reshape(NC, N, HALF)
    wc, bc = _tc_weights(W1, b1.reshape(1, D), W2, b2.reshape(1, D))
    return _tc_final(dis, s1, x, wc, bc)


# speedup vs baseline: 21.1585x; 1.0905x over previous
import functools

import jax
import jax.numpy as jnp
from jax import lax
from jax.experimental import pallas as pl
from jax.experimental.pallas import tpu as pltpu
from jax.experimental.pallas import tpu_sc as plsc

N = 10000
E = 160000
D = 256
HALF = D // 2
ALPHA = 0.5

NC = 2
NS = 16
NW = NC * NS

EDGES_PER_TILE_DEG = E // NW
EDGES_PER_TILE = E // NS
CHUNK = 125
NCHUNKS = EDGES_PER_TILE // CHUNK
ROWS_PER_TILE = N // NS
NGBUF = 2
NIBUF = 4

_mesh = plsc.VectorSubcoreMesh(core_axis_name="c", subcore_axis_name="s")


@functools.partial(
    pl.kernel,
    out_type=jax.ShapeDtypeStruct((NW, N), jnp.float32),
    mesh=_mesh,
    scratch_types=[
        pltpu.VMEM((EDGES_PER_TILE_DEG,), jnp.int32),
        pltpu.VMEM((N,), jnp.float32),
    ],
    compiler_params=pltpu.CompilerParams(needs_layout_passes=False),
)
def _sc_degree(col_hbm, out_hbm, cidx, hist):
    c = lax.axis_index("c")
    s = lax.axis_index("s")
    wid = s * NC + c

    def zero(i, _):
        hist[pl.ds(i * 16, 16)] = jnp.zeros((16,), jnp.float32)
        return 0

    lax.fori_loop(0, N // 16, zero, 0)

    pltpu.sync_copy(col_hbm.at[pl.ds(wid * EDGES_PER_TILE_DEG, EDGES_PER_TILE_DEG)], cidx)

    ones = jnp.ones((16,), jnp.float32)
    nfull = EDGES_PER_TILE_DEG // 16
    rem = EDGES_PER_TILE_DEG - nfull * 16

    def upd(k, _):
        cv = cidx[pl.ds(k * 16, 16)]
        plsc.addupdate_scatter(hist, [cv], ones)
        return 0

    lax.fori_loop(0, nfull, upd, 0)

    lane = lax.iota(jnp.int32, 16)
    cv = cidx[pl.ds(EDGES_PER_TILE_DEG - 16, 16)]
    keep = lane >= (16 - rem)
    plsc.addupdate_scatter(
        hist,
        [jnp.where(keep, cv, 0)],
        jnp.where(keep, 1.0, 0.0).astype(jnp.float32),
    )

    pltpu.sync_copy(hist, out_hbm.at[wid])


@functools.partial(
    pl.kernel,
    out_type=jax.ShapeDtypeStruct((NC * N, HALF), jnp.float32),
    mesh=_mesh,
    scratch_types=[
        pltpu.VMEM_SHARED((N, HALF), jnp.float32),
        pltpu.VMEM((NGBUF, CHUNK, HALF), jnp.float32),
        pltpu.VMEM((NIBUF, CHUNK), jnp.int32),
        pltpu.VMEM((NCHUNKS, CHUNK), jnp.int32),
    ]
    + [pltpu.SemaphoreType.DMA] * (NGBUF + NIBUF),
    compiler_params=pltpu.CompilerParams(use_tc_tiling_on_sc=False),
)
def _sc_propagate(u_hbm, row2d_hbm, col2d_hbm, out_hbm,
                  slab, gbuf, ribuf, cidx, *sems):
    gsems = sems[:NGBUF]
    isems = sems[NGBUF:]
    c = lax.axis_index("c")
    s = lax.axis_index("s")
    rbase = (c * NS + s) * NCHUNKS

    pltpu.sync_copy(col2d_hbm.at[pl.ds(s * NCHUNKS, NCHUNKS)], cidx)
    pltpu.sync_copy(
        u_hbm.at[pl.ds(c * N + s * ROWS_PER_TILE, ROWS_PER_TILE)],
        slab.at[pl.ds(s * ROWS_PER_TILE, ROWS_PER_TILE)],
    )
    plsc.subcore_barrier()

    def ridx_copy(j, ib):
        return pltpu.make_async_copy(
            row2d_hbm.at[rbase + j], ribuf.at[ib], isems[ib])

    def gather(j, ib, gb):
        return pltpu.make_async_copy(
            u_hbm.at[ribuf.at[ib]], gbuf.at[gb], gsems[gb])

    def scatter(j, gb):
        pltpu.sync_copy(gbuf.at[gb], slab.at[cidx.at[j]], add=True)

    for ib in range(NIBUF):
        ridx_copy(ib, ib).start()
    for j in range(NGBUF):
        ridx_copy(j, j).wait()
        gather(j, j, j).start()

    def body(jj, _):
        g = NIBUF * jj
        for b in range(NIBUF):
            j = g + b
            gb = b % NGBUF
            gather(j, b, gb).wait()

            @pl.when(j + NIBUF < NCHUNKS)
            def _():
                ridx_copy(j + NIBUF, b).start()

            scatter(j, gb)

            @pl.when(j + NGBUF < NCHUNKS)
            def _():
                nb = (b + NGBUF) % NIBUF
                ridx_copy(j + NGBUF, nb).wait()
                gather(j + NGBUF, nb, gb).start()

        return 0

    lax.fori_loop(0, NCHUNKS // NIBUF, body, 0)
    plsc.subcore_barrier()

    pltpu.sync_copy(
        slab.at[pl.ds(s * ROWS_PER_TILE, ROWS_PER_TILE)],
        out_hbm.at[pl.ds(c * N + s * ROWS_PER_TILE, ROWS_PER_TILE)],
    )


_BLK = 2000
_GRID = N // _BLK


def _dis_block(degp_ref, dis_s):
    @pl.when(pl.program_id(0) == 0)
    def _():
        deg = jnp.sum(degp_ref[...], axis=0) + 1.0
        dis_s[...] = lax.rsqrt(deg).reshape(N, 1)

    return dis_s[pl.ds(pl.program_id(0) * _BLK, _BLK), :]


def _tc_scale0_body(degp_ref, x_ref, out_ref, dis_s):
    dis = _dis_block(degp_ref, dis_s)
    xb = x_ref[...]
    out_ref[...] = jnp.stack(
        [dis * xb[:, :HALF], dis * xb[:, HALF:]], axis=0
    )


def _tc_scale1_body(degp_ref, s_ref, x_ref, out_ref, dis_s):
    dis = _dis_block(degp_ref, dis_s)
    sb = s_ref[...]
    xb = x_ref[...]
    a = (1.0 - ALPHA) * dis * dis
    b = ALPHA * dis
    out_ref[...] = jnp.stack(
        [a * sb[0] + b * xb[:, :HALF], a * sb[1] + b * xb[:, HALF:]], axis=0
    )


def _tc_final_body(degp_ref, s_ref, x_ref, w1_ref, b1_ref, w2_ref, b2_ref,
                   out_ref, dis_s, wc_s, bc_s):
    @pl.when(pl.program_id(0) == 0)
    def _():
        w2 = w2_ref[...]
        wc_s[...] = jnp.dot(w1_ref[...], w2, preferred_element_type=jnp.float32)
        bc_s[...] = (
            jnp.dot(b1_ref[...], w2, preferred_element_type=jnp.float32)
            + b2_ref[...]
        )

    dis = _dis_block(degp_ref, dis_s)
    sb = s_ref[...]
    xb = x_ref[...]
    agg = jnp.concatenate([sb[0], sb[1]], axis=-1)
    y = jax.nn.relu((1.0 - ALPHA) * dis * agg + ALPHA * xb)
    out_ref[...] = (
        jnp.dot(y, wc_s[...], preferred_element_type=jnp.float32) + bc_s[...]
    )


_degp_spec = pl.BlockSpec((NW, N), lambda i: (0, 0))
_x_spec = pl.BlockSpec((_BLK, D), lambda i: (i, 0))
_s_spec = pl.BlockSpec((NC, _BLK, HALF), lambda i: (0, i, 0))
_w_spec = pl.BlockSpec((D, D), lambda i: (0, 0))
_b_spec = pl.BlockSpec((1, D), lambda i: (0, 0))
_dis_scratch = pltpu.VMEM((N, 1), jnp.float32)


def _tc_scale0(degp, x):
    return pl.pallas_call(
        _tc_scale0_body,
        grid=(_GRID,),
        in_specs=[_degp_spec, _x_spec],
        out_specs=_s_spec,
        out_shape=jax.ShapeDtypeStruct((NC, N, HALF), jnp.float32),
        scratch_shapes=[_dis_scratch],
    )(degp, x)


def _tc_scale1(degp, s, x):
    return pl.pallas_call(
        _tc_scale1_body,
        grid=(_GRID,),
        in_specs=[_degp_spec, _s_spec, _x_spec],
        out_specs=_s_spec,
        out_shape=jax.ShapeDtypeStruct((NC, N, HALF), jnp.float32),
        scratch_shapes=[_dis_scratch],
    )(degp, s, x)


def _tc_final(degp, s, x, w1, b1, w2, b2):
    return pl.pallas_call(
        _tc_final_body,
        grid=(_GRID,),
        in_specs=[
            _degp_spec, _s_spec, _x_spec,
            _w_spec, _b_spec, _w_spec, _b_spec,
        ],
        out_specs=_x_spec,
        out_shape=jax.ShapeDtypeStruct((N, D), jnp.float32),
        scratch_shapes=[
            _dis_scratch,
            pltpu.VMEM((D, D), jnp.float32),
            pltpu.VMEM((1, D), jnp.float32),
        ],
    )(degp, s, x, w1, b1, w2, b2)


def kernel(x, edge_index, W1, b1, W2, b2):
    row = edge_index[0]
    col = edge_index[1]

    row2d = jnp.concatenate([row, row + N]).reshape(NC * E // CHUNK, CHUNK)
    col2d = col.reshape(E // CHUNK, CHUNK)

    degp = _sc_degree(col)
    u0 = _tc_scale0(degp, x).reshape(NC * N, HALF)
    s0 = _sc_propagate(u0, row2d, col2d).reshape(NC, N, HALF)
    u1 = _tc_scale1(degp, s0, x).reshape(NC * N, HALF)
    s1 = _sc_propagate(u1, row2d, col2d).reshape(NC, N, HALF)
    return _tc_final(degp, s1, x, W1, b1.reshape(1, D), W2, b2.reshape(1, D))


# speedup vs baseline: 21.5361x; 1.0178x over previous
import functools

import jax
import jax.numpy as jnp
from jax import lax
from jax.experimental import pallas as pl
from jax.experimental.pallas import tpu as pltpu
from jax.experimental.pallas import tpu_sc as plsc

N = 10000
E = 160000
D = 256
HALF = D // 2
ALPHA = 0.5

NC = 2
NS = 16
NW = NC * NS

EDGES_PER_TILE_DEG = E // NW
EDGES_PER_TILE = E // NS
CHUNK = 125
NCHUNKS = EDGES_PER_TILE // CHUNK
ROWS_PER_TILE = N // NS
NGBUF = 2
NIBUF = 4

_mesh = plsc.VectorSubcoreMesh(core_axis_name="c", subcore_axis_name="s")


@functools.partial(
    pl.kernel,
    out_type=jax.ShapeDtypeStruct((NW, N), jnp.float32),
    mesh=_mesh,
    scratch_types=[
        pltpu.VMEM((EDGES_PER_TILE_DEG,), jnp.int32),
        pltpu.VMEM((N,), jnp.float32),
    ],
    compiler_params=pltpu.CompilerParams(needs_layout_passes=False),
)
def _sc_degree(col_hbm, out_hbm, cidx, hist):
    c = lax.axis_index("c")
    s = lax.axis_index("s")
    wid = s * NC + c

    def zero(i, _):
        hist[pl.ds(i * 16, 16)] = jnp.zeros((16,), jnp.float32)
        return 0

    lax.fori_loop(0, N // 16, zero, 0)

    pltpu.sync_copy(col_hbm.at[pl.ds(wid * EDGES_PER_TILE_DEG, EDGES_PER_TILE_DEG)], cidx)

    ones = jnp.ones((16,), jnp.float32)
    nfull = EDGES_PER_TILE_DEG // 16
    rem = EDGES_PER_TILE_DEG - nfull * 16

    def upd(k, _):
        cv = cidx[pl.ds(k * 16, 16)]
        plsc.addupdate_scatter(hist, [cv], ones)
        return 0

    lax.fori_loop(0, nfull, upd, 0)

    lane = lax.iota(jnp.int32, 16)
    cv = cidx[pl.ds(EDGES_PER_TILE_DEG - 16, 16)]
    keep = lane >= (16 - rem)
    plsc.addupdate_scatter(
        hist,
        [jnp.where(keep, cv, 0)],
        jnp.where(keep, 1.0, 0.0).astype(jnp.float32),
    )

    pltpu.sync_copy(hist, out_hbm.at[wid])


@functools.partial(
    pl.kernel,
    out_type=jax.ShapeDtypeStruct((NC * N, HALF), jnp.float32),
    mesh=_mesh,
    scratch_types=[
        pltpu.VMEM_SHARED((N, HALF), jnp.float32),
        pltpu.VMEM((NGBUF, CHUNK, HALF), jnp.float32),
        pltpu.VMEM((NIBUF, CHUNK), jnp.int32),
        pltpu.VMEM((NCHUNKS, CHUNK), jnp.int32),
    ]
    + [pltpu.SemaphoreType.DMA] * (NGBUF + NIBUF + 1),
    compiler_params=pltpu.CompilerParams(use_tc_tiling_on_sc=False),
)
def _sc_propagate(u_hbm, row2d_hbm, col2d_hbm, out_hbm,
                  slab, gbuf, ribuf, cidx, *sems):
    gsems = sems[:NGBUF]
    isems = sems[NGBUF:NGBUF + NIBUF]
    initsem = sems[NGBUF + NIBUF]
    c = lax.axis_index("c")
    s = lax.axis_index("s")
    rbase = (c * NS + s) * NCHUNKS

    pltpu.sync_copy(col2d_hbm.at[pl.ds(s * NCHUNKS, NCHUNKS)], cidx)
    init = pltpu.make_async_copy(
        u_hbm.at[pl.ds(c * N + s * ROWS_PER_TILE, ROWS_PER_TILE)],
        slab.at[pl.ds(s * ROWS_PER_TILE, ROWS_PER_TILE)],
        initsem,
    )
    init.start()

    def ridx_copy(j, ib):
        return pltpu.make_async_copy(
            row2d_hbm.at[rbase + j], ribuf.at[ib], isems[ib])

    def gather(j, ib, gb):
        return pltpu.make_async_copy(
            u_hbm.at[ribuf.at[ib]], gbuf.at[gb], gsems[gb])

    def scatter(j, gb):
        pltpu.sync_copy(gbuf.at[gb], slab.at[cidx.at[j]], add=True)

    for ib in range(NIBUF):
        ridx_copy(ib, ib).start()
    for j in range(NGBUF):
        ridx_copy(j, j).wait()
        gather(j, j, j).start()

    init.wait()
    plsc.subcore_barrier()

    def body(jj, _):
        g = NIBUF * jj
        for b in range(NIBUF):
            j = g + b
            gb = b % NGBUF
            gather(j, b, gb).wait()

            @pl.when(j + NIBUF < NCHUNKS)
            def _():
                ridx_copy(j + NIBUF, b).start()

            scatter(j, gb)

            @pl.when(j + NGBUF < NCHUNKS)
            def _():
                nb = (b + NGBUF) % NIBUF
                ridx_copy(j + NGBUF, nb).wait()
                gather(j + NGBUF, nb, gb).start()

        return 0

    lax.fori_loop(0, NCHUNKS // NIBUF, body, 0)
    plsc.subcore_barrier()

    pltpu.sync_copy(
        slab.at[pl.ds(s * ROWS_PER_TILE, ROWS_PER_TILE)],
        out_hbm.at[pl.ds(c * N + s * ROWS_PER_TILE, ROWS_PER_TILE)],
    )


_BLK = 2000
_GRID = N // _BLK


def _dis_block(degp_ref, dis_s):
    @pl.when(pl.program_id(0) == 0)
    def _():
        deg = jnp.sum(degp_ref[...], axis=0) + 1.0
        dis_s[...] = lax.rsqrt(deg).reshape(N, 1)

    return dis_s[pl.ds(pl.program_id(0) * _BLK, _BLK), :]


def _tc_scale0_body(degp_ref, x_ref, out_ref, dis_s):
    dis = _dis_block(degp_ref, dis_s)
    xb = x_ref[...]
    out_ref[...] = jnp.stack(
        [dis * xb[:, :HALF], dis * xb[:, HALF:]], axis=0
    )


def _tc_scale1_body(degp_ref, s_ref, x_ref, out_ref, dis_s):
    dis = _dis_block(degp_ref, dis_s)
    sb = s_ref[...]
    xb = x_ref[...]
    a = (1.0 - ALPHA) * dis * dis
    b = ALPHA * dis
    out_ref[...] = jnp.stack(
        [a * sb[0] + b * xb[:, :HALF], a * sb[1] + b * xb[:, HALF:]], axis=0
    )


def _tc_final_body(degp_ref, s_ref, x_ref, w1_ref, b1_ref, w2_ref, b2_ref,
                   out_ref, dis_s, wc_s, bc_s):
    @pl.when(pl.program_id(0) == 0)
    def _():
        w2 = w2_ref[...]
        wc_s[...] = jnp.dot(w1_ref[...], w2, preferred_element_type=jnp.float32)
        bc_s[...] = (
            jnp.dot(b1_ref[...], w2, preferred_element_type=jnp.float32)
            + b2_ref[...]
        )

    dis = _dis_block(degp_ref, dis_s)
    sb = s_ref[...]
    xb = x_ref[...]
    agg = jnp.concatenate([sb[0], sb[1]], axis=-1)
    y = jax.nn.relu((1.0 - ALPHA) * dis * agg + ALPHA * xb)
    out_ref[...] = (
        jnp.dot(y, wc_s[...], preferred_element_type=jnp.float32) + bc_s[...]
    )


_degp_spec = pl.BlockSpec((NW, N), lambda i: (0, 0))
_x_spec = pl.BlockSpec((_BLK, D), lambda i: (i, 0))
_s_spec = pl.BlockSpec((NC, _BLK, HALF), lambda i: (0, i, 0))
_w_spec = pl.BlockSpec((D, D), lambda i: (0, 0))
_b_spec = pl.BlockSpec((1, D), lambda i: (0, 0))
_dis_scratch = pltpu.VMEM((N, 1), jnp.float32)


def _tc_scale0(degp, x):
    return pl.pallas_call(
        _tc_scale0_body,
        grid=(_GRID,),
        in_specs=[_degp_spec, _x_spec],
        out_specs=_s_spec,
        out_shape=jax.ShapeDtypeStruct((NC, N, HALF), jnp.float32),
        scratch_shapes=[_dis_scratch],
    )(degp, x)


def _tc_scale1(degp, s, x):
    return pl.pallas_call(
        _tc_scale1_body,
        grid=(_GRID,),
        in_specs=[_degp_spec, _s_spec, _x_spec],
        out_specs=_s_spec,
        out_shape=jax.ShapeDtypeStruct((NC, N, HALF), jnp.float32),
        scratch_shapes=[_dis_scratch],
    )(degp, s, x)


def _tc_final(degp, s, x, w1, b1, w2, b2):
    return pl.pallas_call(
        _tc_final_body,
        grid=(_GRID,),
        in_specs=[
            _degp_spec, _s_spec, _x_spec,
            _w_spec, _b_spec, _w_spec, _b_spec,
        ],
        out_specs=_x_spec,
        out_shape=jax.ShapeDtypeStruct((N, D), jnp.float32),
        scratch_shapes=[
            _dis_scratch,
            pltpu.VMEM((D, D), jnp.float32),
            pltpu.VMEM((1, D), jnp.float32),
        ],
    )(degp, s, x, w1, b1, w2, b2)


def kernel(x, edge_index, W1, b1, W2, b2):
    row = edge_index[0]
    col = edge_index[1]

    row2d = jnp.concatenate([row, row + N]).reshape(NC * E // CHUNK, CHUNK)
    col2d = col.reshape(E // CHUNK, CHUNK)

    degp = _sc_degree(col)
    u0 = _tc_scale0(degp, x).reshape(NC * N, HALF)
    s0 = _sc_propagate(u0, row2d, col2d).reshape(NC, N, HALF)
    u1 = _tc_scale1(degp, s0, x).reshape(NC * N, HALF)
    s1 = _sc_propagate(u1, row2d, col2d).reshape(NC, N, HALF)
    return _tc_final(degp, s1, x, W1, b1.reshape(1, D), W2, b2.reshape(1, D))


# speedup vs baseline: 22.9865x; 1.0673x over previous
import functools

import jax
import jax.numpy as jnp
from jax import lax
from jax.experimental import pallas as pl
from jax.experimental.pallas import tpu as pltpu
from jax.experimental.pallas import tpu_sc as plsc

N = 10000
E = 160000
D = 256
HALF = D // 2
ALPHA = 0.5

NC = 2
NS = 16
NW = NC * NS

EDGES_PER_TILE_DEG = E // NW
EDGES_PER_TILE = E // NS
CHUNK = 125
NCHUNKS = EDGES_PER_TILE // CHUNK
ROWS_PER_TILE = N // NS
NBUF = 3

_mesh = plsc.VectorSubcoreMesh(core_axis_name="c", subcore_axis_name="s")


@functools.partial(
    pl.kernel,
    out_type=jax.ShapeDtypeStruct((NW, N), jnp.float32),
    mesh=_mesh,
    scratch_types=[
        pltpu.VMEM((EDGES_PER_TILE_DEG,), jnp.int32),
        pltpu.VMEM((N,), jnp.float32),
    ],
    compiler_params=pltpu.CompilerParams(needs_layout_passes=False),
)
def _sc_degree(col_hbm, out_hbm, cidx, hist):
    c = lax.axis_index("c")
    s = lax.axis_index("s")
    wid = s * NC + c

    def zero(i, _):
        hist[pl.ds(i * 16, 16)] = jnp.zeros((16,), jnp.float32)
        return 0

    lax.fori_loop(0, N // 16, zero, 0)

    pltpu.sync_copy(col_hbm.at[pl.ds(wid * EDGES_PER_TILE_DEG, EDGES_PER_TILE_DEG)], cidx)

    ones = jnp.ones((16,), jnp.float32)
    nfull = EDGES_PER_TILE_DEG // 16
    rem = EDGES_PER_TILE_DEG - nfull * 16

    def upd(k, _):
        cv = cidx[pl.ds(k * 16, 16)]
        plsc.addupdate_scatter(hist, [cv], ones)
        return 0

    lax.fori_loop(0, nfull, upd, 0)

    lane = lax.iota(jnp.int32, 16)
    cv = cidx[pl.ds(EDGES_PER_TILE_DEG - 16, 16)]
    keep = lane >= (16 - rem)
    plsc.addupdate_scatter(
        hist,
        [jnp.where(keep, cv, 0)],
        jnp.where(keep, 1.0, 0.0).astype(jnp.float32),
    )

    pltpu.sync_copy(hist, out_hbm.at[wid])


@functools.partial(
    pl.kernel,
    out_type=jax.ShapeDtypeStruct((NC * N, HALF), jnp.float32),
    mesh=_mesh,
    scratch_types=[
        pltpu.VMEM_SHARED((N, HALF), jnp.float32),
        pltpu.VMEM((NBUF, CHUNK, HALF), jnp.float32),
        pltpu.VMEM((NBUF, CHUNK), jnp.int32),
        pltpu.VMEM((NBUF, CHUNK), jnp.int32),
    ]
    + [pltpu.SemaphoreType.DMA] * (4 * NBUF + 1),
    compiler_params=pltpu.CompilerParams(use_tc_tiling_on_sc=False),
)
def _sc_propagate(u_hbm, row2d_hbm, col2d_hbm, out_hbm,
                  slab, gbuf, ribuf, cibuf, *sems):
    gsems = sems[:NBUF]
    rsems = sems[NBUF:2 * NBUF]
    csems = sems[2 * NBUF:3 * NBUF]
    ssems = sems[3 * NBUF:4 * NBUF]
    initsem = sems[4 * NBUF]
    c = lax.axis_index("c")
    s = lax.axis_index("s")
    rbase = (c * NS + s) * NCHUNKS
    cbase = s * NCHUNKS

    init = pltpu.make_async_copy(
        u_hbm.at[pl.ds(c * N + s * ROWS_PER_TILE, ROWS_PER_TILE)],
        slab.at[pl.ds(s * ROWS_PER_TILE, ROWS_PER_TILE)],
        initsem,
    )
    init.start()

    def ridx_copy(j, b):
        return pltpu.make_async_copy(row2d_hbm.at[rbase + j], ribuf.at[b],
                                     rsems[b])

    def cidx_copy(j, b):
        return pltpu.make_async_copy(col2d_hbm.at[cbase + j], cibuf.at[b],
                                     csems[b])

    def gather(b):
        return pltpu.make_async_copy(u_hbm.at[ribuf.at[b]], gbuf.at[b],
                                     gsems[b])

    def scatter_start(b):
        pltpu.async_copy(gbuf.at[b], slab.at[cibuf.at[b]], ssems[b], add=True)

    def scatter_wait(b):
        pltpu.make_async_copy(gbuf.at[b], slab.at[cibuf.at[b]],
                              ssems[b]).wait()

    def step(j, b, bp, first=False, ncidx=True, nridx=True, ngather=True):
        gather(b).wait()
        cidx_copy(j, b).wait()
        scatter_start(b)
        if not first:
            scatter_wait(bp)
        if ncidx:
            cidx_copy(j + 2, bp).start()
        if nridx:
            ridx_copy(j + 3, b).start()
        if ngather:
            ridx_copy(j + 2, bp).wait()
            gather(bp).start()

    for k in range(NBUF):
        ridx_copy(k, k).start()
    for k in range(2):
        cidx_copy(k, k).start()
    for k in range(2):
        ridx_copy(k, k).wait()
        gather(k).start()

    init.wait()
    plsc.subcore_barrier()

    step(0, 0, 2, first=True)
    step(1, 1, 0)

    def body(jj, _):
        j = 2 + 3 * jj
        step(j, 2, 1)
        step(j + 1, 0, 2)
        step(j + 2, 1, 0)
        return 0

    lax.fori_loop(0, (NCHUNKS - 5) // NBUF, body, 0)

    step(NCHUNKS - 3, 2, 1, nridx=False)
    step(NCHUNKS - 2, 0, 2, ncidx=False, nridx=False, ngather=False)
    step(NCHUNKS - 1, 1, 0, ncidx=False, nridx=False, ngather=False)
    scatter_wait(1)
    plsc.subcore_barrier()

    pltpu.sync_copy(
        slab.at[pl.ds(s * ROWS_PER_TILE, ROWS_PER_TILE)],
        out_hbm.at[pl.ds(c * N + s * ROWS_PER_TILE, ROWS_PER_TILE)],
    )


_BLK = 2000
_GRID = N // _BLK


def _dis_block(degp_ref, dis_s):
    @pl.when(pl.program_id(0) == 0)
    def _():
        deg = jnp.sum(degp_ref[...], axis=0) + 1.0
        dis_s[...] = lax.rsqrt(deg).reshape(N, 1)

    return dis_s[pl.ds(pl.program_id(0) * _BLK, _BLK), :]


def _tc_scale0_body(degp_ref, x_ref, out_ref, dis_s):
    dis = _dis_block(degp_ref, dis_s)
    xb = x_ref[...]
    out_ref[...] = jnp.stack(
        [dis * xb[:, :HALF], dis * xb[:, HALF:]], axis=0
    )


def _tc_scale1_body(degp_ref, s_ref, x_ref, out_ref, dis_s):
    dis = _dis_block(degp_ref, dis_s)
    sb = s_ref[...]
    xb = x_ref[...]
    a = (1.0 - ALPHA) * dis * dis
    b = ALPHA * dis
    out_ref[...] = jnp.stack(
        [a * sb[0] + b * xb[:, :HALF], a * sb[1] + b * xb[:, HALF:]], axis=0
    )


def _tc_final_body(degp_ref, s_ref, x_ref, w1_ref, b1_ref, w2_ref, b2_ref,
                   out_ref, dis_s, wc_s, bc_s):
    @pl.when(pl.program_id(0) == 0)
    def _():
        w2 = w2_ref[...]
        wc_s[...] = jnp.dot(w1_ref[...], w2, preferred_element_type=jnp.float32)
        bc_s[...] = (
            jnp.dot(b1_ref[...], w2, preferred_element_type=jnp.float32)
            + b2_ref[...]
        )

    dis = _dis_block(degp_ref, dis_s)
    sb = s_ref[...]
    xb = x_ref[...]
    agg = jnp.concatenate([sb[0], sb[1]], axis=-1)
    y = jax.nn.relu((1.0 - ALPHA) * dis * agg + ALPHA * xb)
    out_ref[...] = (
        jnp.dot(y, wc_s[...], preferred_element_type=jnp.float32) + bc_s[...]
    )


_degp_spec = pl.BlockSpec((NW, N), lambda i: (0, 0))
_x_spec = pl.BlockSpec((_BLK, D), lambda i: (i, 0))
_s_spec = pl.BlockSpec((NC, _BLK, HALF), lambda i: (0, i, 0))
_w_spec = pl.BlockSpec((D, D), lambda i: (0, 0))
_b_spec = pl.BlockSpec((1, D), lambda i: (0, 0))
_dis_scratch = pltpu.VMEM((N, 1), jnp.float32)


def _tc_scale0(degp, x):
    return pl.pallas_call(
        _tc_scale0_body,
        grid=(_GRID,),
        in_specs=[_degp_spec, _x_spec],
        out_specs=_s_spec,
        out_shape=jax.ShapeDtypeStruct((NC, N, HALF), jnp.float32),
        scratch_shapes=[_dis_scratch],
    )(degp, x)


def _tc_scale1(degp, s, x):
    return pl.pallas_call(
        _tc_scale1_body,
        grid=(_GRID,),
        in_specs=[_degp_spec, _s_spec, _x_spec],
        out_specs=_s_spec,
        out_shape=jax.ShapeDtypeStruct((NC, N, HALF), jnp.float32),
        scratch_shapes=[_dis_scratch],
    )(degp, s, x)


def _tc_final(degp, s, x, w1, b1, w2, b2):
    return pl.pallas_call(
        _tc_final_body,
        grid=(_GRID,),
        in_specs=[
            _degp_spec, _s_spec, _x_spec,
            _w_spec, _b_spec, _w_spec, _b_spec,
        ],
        out_specs=_x_spec,
        out_shape=jax.ShapeDtypeStruct((N, D), jnp.float32),
        scratch_shapes=[
            _dis_scratch,
            pltpu.VMEM((D, D), jnp.float32),
            pltpu.VMEM((1, D), jnp.float32),
        ],
    )(degp, s, x, w1, b1, w2, b2)


def kernel(x, edge_index, W1, b1, W2, b2):
    row = edge_index[0]
    col = edge_index[1]

    row2d = jnp.concatenate([row, row + N]).reshape(NC * E // CHUNK, CHUNK)
    col2d = col.reshape(E // CHUNK, CHUNK)

    degp = _sc_degree(col)
    u0 = _tc_scale0(degp, x).reshape(NC * N, HALF)
    s0 = _sc_propagate(u0, row2d, col2d).reshape(NC, N, HALF)
    u1 = _tc_scale1(degp, s0, x).reshape(NC * N, HALF)
    s1 = _sc_propagate(u1, row2d, col2d).reshape(NC, N, HALF)
    return _tc_final(degp, s1, x, W1, b1.reshape(1, D), W2, b2.reshape(1, D))
